# baseline (speedup 1.0000x reference)
import functools

import jax
import jax.numpy as jnp
from jax import lax
from jax.experimental import pallas as pl
from jax.experimental.pallas import tpu as pltpu
from jax.experimental.pallas import tpu_sc as plsc

NC = 2
NS = 16
NW = NC * NS
CH = 128
F = 128


def _sc_mesh():
    return plsc.VectorSubcoreMesh(core_axis_name="c", subcore_axis_name="s")


def _make_deg(N, E):
    nchunk = E // CH
    assert E % CH == 0 and N % 16 == 0
    nzc = N // 16

    @functools.partial(
        pl.kernel,
        out_type=jax.ShapeDtypeStruct((NC, N), jnp.float32),
        mesh=_sc_mesh(),
        scratch_types=[
            pltpu.VMEM_SHARED((N,), jnp.float32),
            pltpu.VMEM((2, CH), jnp.int32),
            pltpu.VMEM((2, CH), jnp.float32),
            pltpu.VMEM((16,), jnp.float32),
            pltpu.SemaphoreType.DMA,
            pltpu.SemaphoreType.DMA,
        ],
    )
    def degk(dst_h, ew_h, out_h, acc, dst_b, ew_b, zb, sem0, sem1):
        cid = lax.axis_index("c")
        sid = lax.axis_index("s")
        wid = sid * NC + cid
        zb[...] = jnp.zeros((16,), jnp.float32)

        nz = (nzc - 1 - sid) // NS + 1

        def zbody(i, _):
            r0 = (sid + i * NS) * 16
            pltpu.sync_copy(zb, acc.at[pl.ds(r0, 16)])
            return 0

        lax.fori_loop(0, nz, zbody, 0)
        plsc.subcore_barrier()

        ne = (nchunk - 1 - wid) // NW + 1
        sems = (sem0, sem1)

        def load_idx(k, b):
            off = (wid + k * NW) * CH
            pltpu.async_copy(dst_h.at[pl.ds(off, CH)], dst_b.at[b], sems[b])
            pltpu.async_copy(ew_h.at[pl.ds(off, CH)], ew_b.at[b], sems[b])

        def wait_idx(k, b):
            off = (wid + k * NW) * CH
            pltpu.make_async_copy(dst_h.at[pl.ds(off, CH)], dst_b.at[b],
                                  sems[b]).wait()
            pltpu.make_async_copy(ew_h.at[pl.ds(off, CH)], ew_b.at[b],
                                  sems[b]).wait()

        def consume(k, b):
            wait_idx(k, b)
            pltpu.sync_copy(ew_b.at[b], acc.at[dst_b.at[b]], add=True)

        @pl.when(ne > 0)
        def _():
            load_idx(0, 0)

        def pair(jj, _):
            k0 = 2 * jj
            k1 = k0 + 1

            @pl.when(k1 < ne)
            def _():
                load_idx(k1, 1)

            @pl.when(k0 < ne)
            def _():
                consume(k0, 0)

            @pl.when(k1 + 1 < ne)
            def _():
                load_idx(k1 + 1, 0)

            @pl.when(k1 < ne)
            def _():
                consume(k1, 1)

            return 0

        lax.fori_loop(0, (ne + 1) // 2, pair, 0)
        plsc.subcore_barrier()

        def obody(i, _):
            r0 = (sid + i * NS) * 16
            pltpu.sync_copy(acc.at[pl.ds(r0, 16)], zb)
            pltpu.sync_copy(zb, out_h.at[cid, pl.ds(r0, 16)])
            return 0

        lax.fori_loop(0, nz, obody, 0)

    return degk


def _make_agg(N, E):
    nchunk = E // CH
    nzc = N // 16

    @functools.partial(
        pl.kernel,
        out_type=jax.ShapeDtypeStruct((NC, N, F), jnp.float32),
        mesh=_sc_mesh(),
        compiler_params=pltpu.CompilerParams(needs_layout_passes=False),
        scratch_types=[
            pltpu.VMEM_SHARED((N, F), jnp.float32),
            pltpu.VMEM((N,), jnp.float32),
            pltpu.VMEM((2, CH), jnp.int32),
            pltpu.VMEM((2, CH), jnp.int32),
            pltpu.VMEM((2, CH), jnp.float32),
            pltpu.VMEM((CH,), jnp.float32),
            pltpu.VMEM((2, CH, F), jnp.float32),
            pltpu.VMEM((16, F), jnp.float32),
            pltpu.SemaphoreType.DMA,
            pltpu.SemaphoreType.DMA,
        ],
    )
    def aggk(z_h, src_h, dst_h, ew_h, dis_h, out_h,
             acc, dis_v, src_b, dst_b, ew_b, nrm_b, rows, zb, sem0, sem1):
        cid = lax.axis_index("c")
        sid = lax.axis_index("s")
        wid = sid * NC + cid
        pltpu.sync_copy(dis_h, dis_v)

        def zrow(r, _):
            for k in range(F // 16):
                zb[r, pl.ds(k * 16, 16)] = jnp.zeros((16,), jnp.float32)
            return 0

        lax.fori_loop(0, 16, zrow, 0)

        nz = (nzc - 1 - sid) // NS + 1

        def zbody(i, _):
            r0 = (sid + i * NS) * 16
            pltpu.sync_copy(zb, acc.at[pl.ds(r0, 16)])
            return 0

        lax.fori_loop(0, nz, zbody, 0)
        plsc.subcore_barrier()

        ne = (nchunk - 1 - wid) // NW + 1
        sems = (sem0, sem1)

        def load_idx(k, b):
            off = (wid + k * NW) * CH
            pltpu.sync_copy(src_h.at[pl.ds(off, CH)], src_b.at[b])
            pltpu.sync_copy(dst_h.at[pl.ds(off, CH)], dst_b.at[b])
            pltpu.sync_copy(ew_h.at[pl.ds(off, CH)], ew_b.at[b])

        def issue_gather(b):
            pltpu.async_copy(z_h.at[src_b.at[b]], rows.at[b], sems[b])

        def wait_gather(b):
            pltpu.make_async_copy(z_h.at[src_b.at[b]], rows.at[b],
                                  sems[b]).wait()

        def consume(b):
            wait_gather(b)

            def gbody(g, _):
                sl = pl.ds(g * 16, 16)
                si = src_b[b, sl]
                di = dst_b[b, sl]
                w = ew_b[b, sl]
                nrm16 = (plsc.load_gather(dis_v, [si]) * w
                         * plsc.load_gather(dis_v, [di]))
                for r in range(16):
                    s = nrm16[r]
                    row = g * 16 + r
                    for k in range(F // 16):
                        sl2 = pl.ds(k * 16, 16)
                        rows[b, row, sl2] = rows[b, row, sl2] * s
                return 0

            lax.fori_loop(0, CH // 16, gbody, 0)
            pltpu.sync_copy(rows.at[b], acc.at[dst_b.at[b]], add=True)

        @pl.when(ne > 0)
        def _():
            load_idx(0, 0)
            issue_gather(0)

        def pair(jj, _):
            k0 = 2 * jj
            k1 = k0 + 1

            @pl.when(k1 < ne)
            def _():
                load_idx(k1, 1)
                issue_gather(1)

            @pl.when(k0 < ne)
            def _():
                consume(0)

            @pl.when(k1 + 1 < ne)
            def _():
                load_idx(k1 + 1, 0)
                issue_gather(0)

            @pl.when(k1 < ne)
            def _():
                consume(1)

            return 0

        lax.fori_loop(0, (ne + 1) // 2, pair, 0)
        plsc.subcore_barrier()

        def obody(i, _):
            r0 = (sid + i * NS) * 16
            pltpu.sync_copy(acc.at[pl.ds(r0, 16)], zb)
            pltpu.sync_copy(zb, out_h.at[cid, pl.ds(r0, 16)])
            return 0

        lax.fori_loop(0, nz, obody, 0)

    return aggk


def _proj(x2, c2, x_out, W_aa, W_c, W_lm, b_lm, W1):
    N, LM = x_out.shape
    HID = W1.shape[1]
    BLK = 1000
    grid = N // BLK
    hi = lax.Precision.HIGHEST

    def body(x_r, c_r, xo_r, waa_r, wc_r, wlm_r, blm_r, w1_r, z_r):
        x1 = c_r[...] * wc_r[...] + x_r[...] * waa_r[...]
        x1 = x1 + jnp.dot(xo_r[...], wlm_r[...],
                          preferred_element_type=jnp.float32, precision=hi)
        x1 = jnp.maximum(x1 + blm_r[...], 0.0)
        z_r[...] = jnp.dot(x1, w1_r[...],
                           preferred_element_type=jnp.float32, precision=hi)

    return pl.pallas_call(
        body,
        grid=(grid,),
        in_specs=[
            pl.BlockSpec((BLK, 1), lambda i: (i, 0)),
            pl.BlockSpec((BLK, 1), lambda i: (i, 0)),
            pl.BlockSpec((BLK, LM), lambda i: (i, 0)),
            pl.BlockSpec((1, LM), lambda i: (0, 0)),
            pl.BlockSpec((1, LM), lambda i: (0, 0)),
            pl.BlockSpec((LM, LM), lambda i: (0, 0)),
            pl.BlockSpec((LM,), lambda i: (0,)),
            pl.BlockSpec((LM, HID), lambda i: (0, 0)),
        ],
        out_specs=pl.BlockSpec((BLK, HID), lambda i: (i, 0)),
        out_shape=jax.ShapeDtypeStruct((N, HID), jnp.float32),
    )(x2, c2, x_out, W_aa, W_c, W_lm, b_lm, W1)


def _dis(deg_p):
    N = deg_p.shape[1]

    def body(d_r, o_r):
        d = d_r[...]
        o_r[...] = lax.rsqrt(1.0 + d[0] + d[1])

    return pl.pallas_call(
        body,
        out_shape=jax.ShapeDtypeStruct((N,), jnp.float32),
    )(deg_p)


def _mid(p, z1, dis2, b1, W3):
    N = z1.shape[0]
    BLK = 1000
    grid = N // BLK
    hi = lax.Precision.HIGHEST

    def body(p_r, z_r, d_r, b_r, w_r, o_r):
        pv = p_r[...]
        d2 = d_r[...] * d_r[...]
        h = pv[0] + pv[1] + z_r[...] * d2 + b_r[...]
        h = jnp.maximum(h, 0.0)
        o_r[...] = jnp.dot(h, w_r[...],
                           preferred_element_type=jnp.float32, precision=hi)

    return pl.pallas_call(
        body,
        grid=(grid,),
        in_specs=[
            pl.BlockSpec((NC, BLK, F), lambda i: (0, i, 0)),
            pl.BlockSpec((BLK, F), lambda i: (i, 0)),
            pl.BlockSpec((BLK, 1), lambda i: (i, 0)),
            pl.BlockSpec((F,), lambda i: (0,)),
            pl.BlockSpec((F, F), lambda i: (0, 0)),
        ],
        out_specs=pl.BlockSpec((BLK, F), lambda i: (i, 0)),
        out_shape=jax.ShapeDtypeStruct((N, F), jnp.float32),
    )(p, z1, dis2, b1, W3)


def _fin(p, z2, dis2, b3):
    N = z2.shape[0]
    BLK = 1000
    grid = N // BLK

    def body(p_r, z_r, d_r, b_r, o_r):
        pv = p_r[...]
        d2 = d_r[...] * d_r[...]
        o_r[...] = pv[0] + pv[1] + z_r[...] * d2 + b_r[...]

    return pl.pallas_call(
        body,
        grid=(grid,),
        in_specs=[
            pl.BlockSpec((NC, BLK, F), lambda i: (0, i, 0)),
            pl.BlockSpec((BLK, F), lambda i: (i, 0)),
            pl.BlockSpec((BLK, 1), lambda i: (i, 0)),
            pl.BlockSpec((F,), lambda i: (0,)),
        ],
        out_specs=pl.BlockSpec((BLK, F), lambda i: (i, 0)),
        out_shape=jax.ShapeDtypeStruct((N, F), jnp.float32),
    )(p, z2, dis2, b3)


def kernel(x, x_out, edge_index, edge_weight, c, W_aa, W_c, W_lm, b_lm,
           W1, b1, W3, b3):
    N, LM = x_out.shape
    E = edge_index.shape[1]
    src = edge_index[0]
    dst = edge_index[1]

    degk = _make_deg(N, E)
    aggk = _make_agg(N, E)

    deg_p = degk(dst, edge_weight)
    z1 = _proj(x[:, None], c[:, None], x_out, W_aa, W_c, W_lm, b_lm, W1)
    dis = _dis(deg_p)
    dis2 = dis[:, None]

    p1 = aggk(z1, src, dst, edge_weight, dis)
    z2 = _mid(p1, z1, dis2, b1, W3)
    p2 = aggk(z2, src, dst, edge_weight, dis)
    return _fin(p2, z2, dis2, b3)


# speedup vs baseline: 1.5929x; 1.5929x over previous
import functools

import jax
import jax.numpy as jnp
from jax import lax
from jax.experimental import pallas as pl
from jax.experimental.pallas import tpu as pltpu
from jax.experimental.pallas import tpu_sc as plsc

NC = 2
NS = 16
NW = NC * NS
CHD = 128
CH = 64
G = 8
F = 128


def _cdiv(a, b):
    return (a + b - 1) // b


def _sc_mesh():
    return plsc.VectorSubcoreMesh(core_axis_name="c", subcore_axis_name="s")


def _make_deg(N, E):
    nchunk = E // CHD
    assert E % CHD == 0 and N % 16 == 0
    nzc = N // 16

    @functools.partial(
        pl.kernel,
        out_type=jax.ShapeDtypeStruct((NC, N), jnp.float32),
        mesh=_sc_mesh(),
        scratch_types=[
            pltpu.VMEM_SHARED((N,), jnp.float32),
            pltpu.VMEM((2, CHD), jnp.int32),
            pltpu.VMEM((2, CHD), jnp.float32),
            pltpu.VMEM((16,), jnp.float32),
            pltpu.SemaphoreType.DMA,
            pltpu.SemaphoreType.DMA,
        ],
    )
    def degk(dst_h, ew_h, out_h, acc, dst_b, ew_b, zb, sem0, sem1):
        cid = lax.axis_index("c")
        sid = lax.axis_index("s")
        wid = sid * NC + cid
        zb[...] = jnp.zeros((16,), jnp.float32)

        nz = (nzc - 1 - sid) // NS + 1

        def zbody(i, _):
            r0 = (sid + i * NS) * 16
            pltpu.sync_copy(zb, acc.at[pl.ds(r0, 16)])
            return 0

        lax.fori_loop(0, nz, zbody, 0)
        plsc.subcore_barrier()

        ne = (nchunk - 1 - wid) // NW + 1
        sems = (sem0, sem1)

        def load_idx(k, b):
            off = (wid + k * NW) * CHD
            pltpu.async_copy(dst_h.at[pl.ds(off, CHD)], dst_b.at[b], sems[b])
            pltpu.async_copy(ew_h.at[pl.ds(off, CHD)], ew_b.at[b], sems[b])

        def wait_idx(k, b):
            off = (wid + k * NW) * CHD
            pltpu.make_async_copy(dst_h.at[pl.ds(off, CHD)], dst_b.at[b],
                                  sems[b]).wait()
            pltpu.make_async_copy(ew_h.at[pl.ds(off, CHD)], ew_b.at[b],
                                  sems[b]).wait()

        def consume(k, b):
            wait_idx(k, b)
            pltpu.sync_copy(ew_b.at[b], acc.at[dst_b.at[b]], add=True)

        @pl.when(ne > 0)
        def _():
            load_idx(0, 0)

        def pair(jj, _):
            k0 = 2 * jj
            k1 = k0 + 1

            @pl.when(k1 < ne)
            def _():
                load_idx(k1, 1)

            @pl.when(k0 < ne)
            def _():
                consume(k0, 0)

            @pl.when(k1 + 1 < ne)
            def _():
                load_idx(k1 + 1, 0)

            @pl.when(k1 < ne)
            def _():
                consume(k1, 1)

            return 0

        lax.fori_loop(0, (ne + 1) // 2, pair, 0)
        plsc.subcore_barrier()

        def obody(i, _):
            r0 = (sid + i * NS) * 16
            pltpu.sync_copy(acc.at[pl.ds(r0, 16)], zb)
            pltpu.sync_copy(zb, out_h.at[cid, pl.ds(r0, 16)])
            return 0

        lax.fori_loop(0, nz, obody, 0)

    return degk


def _make_agg(N, E):
    nchunk = E // CH
    assert E % CH == 0 and N % 16 == 0 and nchunk % G == 0
    npw = G * _cdiv(nchunk, NW * G)
    nb64 = N // 64
    ntail = N - nb64 * 64

    @functools.partial(
        pl.kernel,
        out_type=jax.ShapeDtypeStruct((NC, N, F), jnp.float32),
        mesh=_sc_mesh(),
        compiler_params=pltpu.CompilerParams(needs_layout_passes=False),
        scratch_types=[
            pltpu.VMEM_SHARED((N, F), jnp.float32),
            pltpu.VMEM((2, G, CH), jnp.int32),
            pltpu.VMEM((2, G, CH), jnp.int32),
            pltpu.VMEM((2, G, CH), jnp.float32),
            pltpu.VMEM((4, CH, F), jnp.float32),
            pltpu.SemaphoreType.DMA,
            pltpu.SemaphoreType.DMA,
            pltpu.SemaphoreType.DMA,
            pltpu.SemaphoreType.DMA,
            pltpu.SemaphoreType.DMA,
            pltpu.SemaphoreType.DMA,
            pltpu.SemaphoreType.DMA,
            pltpu.SemaphoreType.DMA,
            pltpu.SemaphoreType.DMA,
            pltpu.SemaphoreType.DMA,
        ],
    )
    def aggk(z_h, src2_h, dst2_h, ew2_h, out_h,
             acc, src_b, dst_b, ew_b, rows,
             gs0, gs1, gs2, gs3, ss0, ss1, ss2, ss3, is0, is1):
        cid = lax.axis_index("c")
        sid = lax.axis_index("s")
        wid = sid * NC + cid
        gsems = (gs0, gs1, gs2, gs3)
        ssems = (ss0, ss1, ss2, ss3)
        isems = (is0, is1)

        s0 = wid * npw
        cnt = jnp.clip(nchunk - s0, 0, npw)

        def issue_group(K, ib):
            c0 = s0 + K * G
            pltpu.async_copy(src2_h.at[pl.ds(c0, G)], src_b.at[ib],
                             isems[ib])
            pltpu.async_copy(dst2_h.at[pl.ds(c0, G)], dst_b.at[ib],
                             isems[ib])
            pltpu.async_copy(ew2_h.at[pl.ds(c0, G)], ew_b.at[ib],
                             isems[ib])

        def wait_group(K, ib):
            c0 = s0 + K * G
            pltpu.make_async_copy(src2_h.at[pl.ds(c0, G)], src_b.at[ib],
                                  isems[ib]).wait()
            pltpu.make_async_copy(dst2_h.at[pl.ds(c0, G)], dst_b.at[ib],
                                  isems[ib]).wait()
            pltpu.make_async_copy(ew2_h.at[pl.ds(c0, G)], ew_b.at[ib],
                                  isems[ib]).wait()

        @pl.when(cnt > 0)
        def _():
            issue_group(0, 0)

        def zrow(r, _):
            for k in range(F // 16):
                rows[0, r, pl.ds(k * 16, 16)] = jnp.zeros((16,), jnp.float32)
            return 0

        lax.fori_loop(0, CH, zrow, 0)

        nz = (nb64 - 1 - sid) // NS + 1

        def zbody(i, _):
            r0 = (sid + i * NS) * 64
            pltpu.sync_copy(rows.at[0], acc.at[pl.ds(r0, 64)])
            return 0

        lax.fori_loop(0, nz, zbody, 0)
        if ntail:
            @pl.when(sid == 0)
            def _():
                pltpu.sync_copy(rows.at[0, pl.ds(0, ntail)],
                                acc.at[pl.ds(nb64 * 64, ntail)])

        plsc.subcore_barrier()

        def issue_gather(c, b):
            K = c // G
            jj = c - K * G
            ib = K % 2
            pltpu.async_copy(z_h.at[src_b.at[ib, jj]], rows.at[b], gsems[b])

        def wait_gather(c, b):
            K = c // G
            jj = c - K * G
            ib = K % 2
            pltpu.make_async_copy(z_h.at[src_b.at[ib, jj]], rows.at[b],
                                  gsems[b]).wait()

        def wait_scatter(b):
            pltpu.make_async_copy(rows.at[b], acc.at[dst_b.at[0, 0]],
                                  ssems[b]).wait()

        def consume(c, b):
            K = c // G
            jj = c - K * G
            ib = K % 2
            wait_gather(c, b)

            def gbody(g, _):
                sl = pl.ds(g * 16, 16)
                w16 = ew_b[ib, jj, sl]
                for r in range(16):
                    s = w16[r]
                    row = g * 16 + r
                    for k in range(F // 16):
                        sl2 = pl.ds(k * 16, 16)
                        rows[b, row, sl2] = rows[b, row, sl2] * s
                return 0

            lax.fori_loop(0, CH // 16, gbody, 0)
            pltpu.async_copy(rows.at[b], acc.at[dst_b.at[ib, jj]], ssems[b],
                             add=True)

        def prefetch(pc, b):
            K = pc // G
            jj = pc - K * G
            Kp = K % 2

            @pl.when(jj == 0)
            def _():
                @pl.when(Kp == 0)
                def _():
                    wait_group(K, 0)

                @pl.when(Kp == 1)
                def _():
                    wait_group(K, 1)

            @pl.when(jj == 4)
            def _():
                nxt = (K + 1) * G < cnt

                @pl.when(nxt & (Kp == 0))
                def _():
                    issue_group(K + 1, 1)

                @pl.when(nxt & (Kp == 1))
                def _():
                    issue_group(K + 1, 0)

            @pl.when(pc >= 4)
            def _():
                wait_scatter(b)

            issue_gather(pc, b)

        for p in range(3):
            @pl.when(cnt > p)
            def _(p=p):
                prefetch(p, p)

        def quad(kk, _):
            k0 = 4 * kk
            for d in range(4):
                c = k0 + d

                @pl.when(c < cnt)
                def _(c=c, d=d):
                    consume(c, d)

                pc = c + 3
                pb = (d + 3) % 4

                @pl.when(pc < cnt)
                def _(pc=pc, pb=pb):
                    prefetch(pc, pb)

            return 0

        lax.fori_loop(0, (cnt + 3) // 4, quad, 0)
        for b in range(4):
            @pl.when(cnt > b)
            def _(b=b):
                wait_scatter(b)

        plsc.subcore_barrier()

        def obody(i, _):
            r0 = (sid + i * NS) * 64
            pltpu.sync_copy(acc.at[pl.ds(r0, 64)], rows.at[0])
            pltpu.sync_copy(rows.at[0], out_h.at[cid, pl.ds(r0, 64)])
            return 0

        lax.fori_loop(0, nz, obody, 0)
        if ntail:
            @pl.when(sid == 1 % NS)
            def _():
                pltpu.sync_copy(acc.at[pl.ds(nb64 * 64, ntail)],
                                rows.at[1, pl.ds(0, ntail)])
                pltpu.sync_copy(rows.at[1, pl.ds(0, ntail)],
                                out_h.at[cid, pl.ds(nb64 * 64, ntail)])

    return aggk


def _proj(x2, c2, x_out, W_aa, W_c, W_lm, b_lm, W1):
    N, LM = x_out.shape
    HID = W1.shape[1]
    BLK = 1000
    grid = N // BLK
    hi = lax.Precision.HIGHEST

    def body(x_r, c_r, xo_r, waa_r, wc_r, wlm_r, blm_r, w1_r, z_r):
        x1 = c_r[...] * wc_r[...] + x_r[...] * waa_r[...]
        x1 = x1 + jnp.dot(xo_r[...], wlm_r[...],
                          preferred_element_type=jnp.float32, precision=hi)
        x1 = jnp.maximum(x1 + blm_r[...], 0.0)
        z_r[...] = jnp.dot(x1, w1_r[...],
                           preferred_element_type=jnp.float32, precision=hi)

    return pl.pallas_call(
        body,
        grid=(grid,),
        in_specs=[
            pl.BlockSpec((BLK, 1), lambda i: (i, 0)),
            pl.BlockSpec((BLK, 1), lambda i: (i, 0)),
            pl.BlockSpec((BLK, LM), lambda i: (i, 0)),
            pl.BlockSpec((1, LM), lambda i: (0, 0)),
            pl.BlockSpec((1, LM), lambda i: (0, 0)),
            pl.BlockSpec((LM, LM), lambda i: (0, 0)),
            pl.BlockSpec((LM,), lambda i: (0,)),
            pl.BlockSpec((LM, HID), lambda i: (0, 0)),
        ],
        out_specs=pl.BlockSpec((BLK, HID), lambda i: (i, 0)),
        out_shape=jax.ShapeDtypeStruct((N, HID), jnp.float32),
    )(x2, c2, x_out, W_aa, W_c, W_lm, b_lm, W1)


def _dis(deg_p, z1):
    N = deg_p.shape[1]
    BLK = 1000
    grid = N // BLK

    def body(d_r, z_r, dis_r, zs_r):
        d = d_r[...]
        disv = lax.rsqrt(1.0 + d[0] + d[1])
        dis_r[...] = disv
        zs_r[...] = z_r[...] * disv

    return pl.pallas_call(
        body,
        grid=(grid,),
        in_specs=[
            pl.BlockSpec((2, BLK, 1), lambda i: (0, i, 0)),
            pl.BlockSpec((BLK, F), lambda i: (i, 0)),
        ],
        out_specs=[
            pl.BlockSpec((BLK, 1), lambda i: (i, 0)),
            pl.BlockSpec((BLK, F), lambda i: (i, 0)),
        ],
        out_shape=[
            jax.ShapeDtypeStruct((N, 1), jnp.float32),
            jax.ShapeDtypeStruct((N, F), jnp.float32),
        ],
    )(deg_p[:, :, None], z1)


def _mid(p, zs1, dis2, b1, W3):
    N = zs1.shape[0]
    BLK = 1000
    grid = N // BLK
    hi = lax.Precision.HIGHEST

    def body(p_r, z_r, d_r, b_r, w_r, o_r):
        pv = p_r[...]
        disv = d_r[...]
        h = (pv[0] + pv[1] + z_r[...]) * disv + b_r[...]
        h = jnp.maximum(h, 0.0)
        z2 = jnp.dot(h, w_r[...],
                     preferred_element_type=jnp.float32, precision=hi)
        o_r[...] = z2 * disv

    return pl.pallas_call(
        body,
        grid=(grid,),
        in_specs=[
            pl.BlockSpec((NC, BLK, F), lambda i: (0, i, 0)),
            pl.BlockSpec((BLK, F), lambda i: (i, 0)),
            pl.BlockSpec((BLK, 1), lambda i: (i, 0)),
            pl.BlockSpec((F,), lambda i: (0,)),
            pl.BlockSpec((F, F), lambda i: (0, 0)),
        ],
        out_specs=pl.BlockSpec((BLK, F), lambda i: (i, 0)),
        out_shape=jax.ShapeDtypeStruct((N, F), jnp.float32),
    )(p, zs1, dis2, b1, W3)


def _fin(p, zs2, dis2, b3):
    N = zs2.shape[0]
    BLK = 1000
    grid = N // BLK

    def body(p_r, z_r, d_r, b_r, o_r):
        pv = p_r[...]
        o_r[...] = (pv[0] + pv[1] + z_r[...]) * d_r[...] + b_r[...]

    return pl.pallas_call(
        body,
        grid=(grid,),
        in_specs=[
            pl.BlockSpec((NC, BLK, F), lambda i: (0, i, 0)),
            pl.BlockSpec((BLK, F), lambda i: (i, 0)),
            pl.BlockSpec((BLK, 1), lambda i: (i, 0)),
            pl.BlockSpec((F,), lambda i: (0,)),
        ],
        out_specs=pl.BlockSpec((BLK, F), lambda i: (i, 0)),
        out_shape=jax.ShapeDtypeStruct((N, F), jnp.float32),
    )(p, zs2, dis2, b3)


def kernel(x, x_out, edge_index, edge_weight, c, W_aa, W_c, W_lm, b_lm,
           W1, b1, W3, b3):
    N, LM = x_out.shape
    E = edge_index.shape[1]
    src = edge_index[0]
    dst = edge_index[1]
    src2 = src.reshape(E // CH, CH)
    dst2 = dst.reshape(E // CH, CH)
    ew2 = edge_weight.reshape(E // CH, CH)

    degk = _make_deg(N, E)
    aggk = _make_agg(N, E)

    deg_p = degk(dst, edge_weight)
    z1 = _proj(x[:, None], c[:, None], x_out, W_aa, W_c, W_lm, b_lm, W1)
    dis2, zs1 = _dis(deg_p, z1)

    p1 = aggk(zs1, src2, dst2, ew2)
    zs2 = _mid(p1, zs1, dis2, b1, W3)
    p2 = aggk(zs2, src2, dst2, ew2)
    return _fin(p2, zs2, dis2, b3)


# speedup vs baseline: 2.0023x; 1.2570x over previous
import functools

import jax
import jax.numpy as jnp
from jax import lax
from jax.experimental import pallas as pl
from jax.experimental.pallas import tpu as pltpu
from jax.experimental.pallas import tpu_sc as plsc

NC = 2
NS = 16
NW = NC * NS
CHD = 128
CH = 64
G = 8
F = 128


def _cdiv(a, b):
    return (a + b - 1) // b


def _sc_mesh():
    return plsc.VectorSubcoreMesh(core_axis_name="c", subcore_axis_name="s")


def _make_deg(N, E):
    nchunk = E // CHD
    assert E % CHD == 0 and N % 16 == 0
    nzc = N // 16

    @functools.partial(
        pl.kernel,
        out_type=jax.ShapeDtypeStruct((NC, N), jnp.float32),
        mesh=_sc_mesh(),
        scratch_types=[
            pltpu.VMEM_SHARED((N,), jnp.float32),
            pltpu.VMEM((4, CHD), jnp.int32),
            pltpu.VMEM((4, CHD), jnp.float32),
            pltpu.VMEM((16,), jnp.float32),
            pltpu.SemaphoreType.DMA,
            pltpu.SemaphoreType.DMA,
            pltpu.SemaphoreType.DMA,
            pltpu.SemaphoreType.DMA,
            pltpu.SemaphoreType.DMA,
            pltpu.SemaphoreType.DMA,
            pltpu.SemaphoreType.DMA,
            pltpu.SemaphoreType.DMA,
        ],
    )
    def degk(dst_h, ew_h, out_h, acc, dst_b, ew_b, zb,
             ls0, ls1, ls2, ls3, ss0, ss1, ss2, ss3):
        cid = lax.axis_index("c")
        sid = lax.axis_index("s")
        wid = sid * NC + cid
        zb[...] = jnp.zeros((16,), jnp.float32)

        nz = (nzc - 1 - sid) // NS + 1

        def zbody(i, _):
            r0 = (sid + i * NS) * 16
            pltpu.sync_copy(zb, acc.at[pl.ds(r0, 16)])
            return 0

        lax.fori_loop(0, nz, zbody, 0)
        plsc.subcore_barrier()

        ne = (nchunk - 1 - wid) // NW + 1
        lsems = (ls0, ls1, ls2, ls3)
        ssems = (ss0, ss1, ss2, ss3)

        def load_idx(k, b):
            off = (wid + k * NW) * CHD
            pltpu.async_copy(dst_h.at[pl.ds(off, CHD)], dst_b.at[b],
                             lsems[b])
            pltpu.async_copy(ew_h.at[pl.ds(off, CHD)], ew_b.at[b], lsems[b])

        def wait_idx(k, b):
            off = (wid + k * NW) * CHD
            pltpu.make_async_copy(dst_h.at[pl.ds(off, CHD)], dst_b.at[b],
                                  lsems[b]).wait()
            pltpu.make_async_copy(ew_h.at[pl.ds(off, CHD)], ew_b.at[b],
                                  lsems[b]).wait()

        def consume(k, b):
            wait_idx(k, b)
            pltpu.sync_copy(ew_b.at[b], acc.at[dst_b.at[b]], add=True)

        def prefetch(pc, b):
            load_idx(pc, b)

        for p in range(3):
            @pl.when(ne > p)
            def _(p=p):
                prefetch(p, p)

        def quad(kk, _):
            k0 = 4 * kk
            for d in range(4):
                c = k0 + d

                @pl.when(c < ne)
                def _(c=c, d=d):
                    consume(c, d)

                pc = c + 3
                pb = (d + 3) % 4

                @pl.when(pc < ne)
                def _(pc=pc, pb=pb):
                    prefetch(pc, pb)

            return 0

        lax.fori_loop(0, (ne + 3) // 4, quad, 0)
        plsc.subcore_barrier()

        def obody(i, _):
            r0 = (sid + i * NS) * 16
            pltpu.sync_copy(acc.at[pl.ds(r0, 16)], zb)
            pltpu.sync_copy(zb, out_h.at[cid, pl.ds(r0, 16)])
            return 0

        lax.fori_loop(0, nz, obody, 0)

    return degk


def _make_agg(N, E):
    nchunk = E // CH
    assert E % CH == 0 and N % 16 == 0 and nchunk % G == 0
    npw = G * _cdiv(nchunk, NW * G)
    nb64 = N // 64
    ntail = N - nb64 * 64

    @functools.partial(
        pl.kernel,
        out_type=jax.ShapeDtypeStruct((NC, N, F), jnp.float32),
        mesh=_sc_mesh(),
        compiler_params=pltpu.CompilerParams(needs_layout_passes=False),
        scratch_types=[
            pltpu.VMEM_SHARED((N, F), jnp.float32),
            pltpu.VMEM((2, G, CH), jnp.int32),
            pltpu.VMEM((2, G, CH), jnp.int32),
            pltpu.VMEM((2, G, CH), jnp.float32),
            pltpu.VMEM((4, CH, F), jnp.float32),
            pltpu.SemaphoreType.DMA,
            pltpu.SemaphoreType.DMA,
            pltpu.SemaphoreType.DMA,
            pltpu.SemaphoreType.DMA,
            pltpu.SemaphoreType.DMA,
            pltpu.SemaphoreType.DMA,
            pltpu.SemaphoreType.DMA,
            pltpu.SemaphoreType.DMA,
            pltpu.SemaphoreType.DMA,
            pltpu.SemaphoreType.DMA,
        ],
    )
    def aggk(z_h, src2_h, dst2_h, ew2_h, out_h,
             acc, src_b, dst_b, ew_b, rows,
             gs0, gs1, gs2, gs3, ss0, ss1, ss2, ss3, is0, is1):
        cid = lax.axis_index("c")
        sid = lax.axis_index("s")
        wid = sid * NC + cid
        gsems = (gs0, gs1, gs2, gs3)
        ssems = (ss0, ss1, ss2, ss3)
        isems = (is0, is1)

        s0 = wid * npw
        cnt = jnp.clip(nchunk - s0, 0, npw)

        def issue_group(K, ib):
            c0 = s0 + K * G
            pltpu.async_copy(src2_h.at[pl.ds(c0, G)], src_b.at[ib],
                             isems[ib])
            pltpu.async_copy(dst2_h.at[pl.ds(c0, G)], dst_b.at[ib],
                             isems[ib])
            pltpu.async_copy(ew2_h.at[pl.ds(c0, G)], ew_b.at[ib],
                             isems[ib])

        def wait_group(K, ib):
            c0 = s0 + K * G
            pltpu.make_async_copy(src2_h.at[pl.ds(c0, G)], src_b.at[ib],
                                  isems[ib]).wait()
            pltpu.make_async_copy(dst2_h.at[pl.ds(c0, G)], dst_b.at[ib],
                                  isems[ib]).wait()
            pltpu.make_async_copy(ew2_h.at[pl.ds(c0, G)], ew_b.at[ib],
                                  isems[ib]).wait()

        @pl.when(cnt > 0)
        def _():
            issue_group(0, 0)

        def zrow(r, _):
            for k in range(F // 16):
                rows[0, r, pl.ds(k * 16, 16)] = jnp.zeros((16,), jnp.float32)
            return 0

        lax.fori_loop(0, CH, zrow, 0)

        nz = (nb64 - 1 - sid) // NS + 1

        def zbody(i, _):
            r0 = (sid + i * NS) * 64
            pltpu.sync_copy(rows.at[0], acc.at[pl.ds(r0, 64)])
            return 0

        lax.fori_loop(0, nz, zbody, 0)
        if ntail:
            @pl.when(sid == 0)
            def _():
                pltpu.sync_copy(rows.at[0, pl.ds(0, ntail)],
                                acc.at[pl.ds(nb64 * 64, ntail)])

        plsc.subcore_barrier()

        def issue_gather(c, b):
            K = c // G
            jj = c - K * G
            ib = K % 2
            pltpu.async_copy(z_h.at[src_b.at[ib, jj]], rows.at[b], gsems[b])

        def wait_gather(c, b):
            K = c // G
            jj = c - K * G
            ib = K % 2
            pltpu.make_async_copy(z_h.at[src_b.at[ib, jj]], rows.at[b],
                                  gsems[b]).wait()

        def wait_scatter(b):
            pltpu.make_async_copy(rows.at[b], acc.at[dst_b.at[0, 0]],
                                  ssems[b]).wait()

        def consume(c, b):
            K = c // G
            jj = c - K * G
            ib = K % 2
            wait_gather(c, b)

            def gbody(g, _):
                sl = pl.ds(g * 16, 16)
                w16 = ew_b[ib, jj, sl]
                for r in range(16):
                    s = w16[r]
                    row = g * 16 + r
                    for k in range(F // 16):
                        sl2 = pl.ds(k * 16, 16)
                        rows[b, row, sl2] = rows[b, row, sl2] * s
                return 0

            lax.fori_loop(0, CH // 16, gbody, 0)
            pltpu.async_copy(rows.at[b], acc.at[dst_b.at[ib, jj]], ssems[b],
                             add=True)

        def prefetch(pc, b):
            K = pc // G
            jj = pc - K * G
            Kp = K % 2

            @pl.when(jj == 0)
            def _():
                @pl.when(Kp == 0)
                def _():
                    wait_group(K, 0)

                @pl.when(Kp == 1)
                def _():
                    wait_group(K, 1)

            @pl.when(jj == 4)
            def _():
                nxt = (K + 1) * G < cnt

                @pl.when(nxt & (Kp == 0))
                def _():
                    issue_group(K + 1, 1)

                @pl.when(nxt & (Kp == 1))
                def _():
                    issue_group(K + 1, 0)

            @pl.when(pc >= 4)
            def _():
                wait_scatter(b)

            issue_gather(pc, b)

        for p in range(3):
            @pl.when(cnt > p)
            def _(p=p):
                prefetch(p, p)

        def quad(kk, _):
            k0 = 4 * kk
            for d in range(4):
                c = k0 + d

                @pl.when(c < cnt)
                def _(c=c, d=d):
                    consume(c, d)

                pc = c + 3
                pb = (d + 3) % 4

                @pl.when(pc < cnt)
                def _(pc=pc, pb=pb):
                    prefetch(pc, pb)

            return 0

        lax.fori_loop(0, (cnt + 3) // 4, quad, 0)
        for b in range(4):
            @pl.when(cnt > b)
            def _(b=b):
                wait_scatter(b)

        plsc.subcore_barrier()

        def obody(i, _):
            r0 = (sid + i * NS) * 64
            pltpu.sync_copy(acc.at[pl.ds(r0, 64)], rows.at[0])
            pltpu.sync_copy(rows.at[0], out_h.at[cid, pl.ds(r0, 64)])
            return 0

        lax.fori_loop(0, nz, obody, 0)
        if ntail:
            @pl.when(sid == 1 % NS)
            def _():
                pltpu.sync_copy(acc.at[pl.ds(nb64 * 64, ntail)],
                                rows.at[1, pl.ds(0, ntail)])
                pltpu.sync_copy(rows.at[1, pl.ds(0, ntail)],
                                out_h.at[cid, pl.ds(nb64 * 64, ntail)])

    return aggk


def _proj(x2, c2, x_out, W_aa, W_c, W_lm, b_lm, W1):
    N, LM = x_out.shape
    HID = W1.shape[1]
    BLK = 1000
    grid = N // BLK
    hi = lax.Precision.DEFAULT

    def body(x_r, c_r, xo_r, waa_r, wc_r, wlm_r, blm_r, w1_r, z_r):
        x1 = c_r[...] * wc_r[...] + x_r[...] * waa_r[...]
        x1 = x1 + jnp.dot(xo_r[...], wlm_r[...],
                          preferred_element_type=jnp.float32, precision=hi)
        x1 = jnp.maximum(x1 + blm_r[...], 0.0)
        z_r[...] = jnp.dot(x1, w1_r[...],
                           preferred_element_type=jnp.float32, precision=hi)

    return pl.pallas_call(
        body,
        grid=(grid,),
        in_specs=[
            pl.BlockSpec((BLK, 1), lambda i: (i, 0)),
            pl.BlockSpec((BLK, 1), lambda i: (i, 0)),
            pl.BlockSpec((BLK, LM), lambda i: (i, 0)),
            pl.BlockSpec((1, LM), lambda i: (0, 0)),
            pl.BlockSpec((1, LM), lambda i: (0, 0)),
            pl.BlockSpec((LM, LM), lambda i: (0, 0)),
            pl.BlockSpec((LM,), lambda i: (0,)),
            pl.BlockSpec((LM, HID), lambda i: (0, 0)),
        ],
        out_specs=pl.BlockSpec((BLK, HID), lambda i: (i, 0)),
        out_shape=jax.ShapeDtypeStruct((N, HID), jnp.float32),
    )(x2, c2, x_out, W_aa, W_c, W_lm, b_lm, W1)


def _dis(deg_p, z1):
    N = deg_p.shape[1]
    BLK = 1000
    grid = N // BLK

    def body(d_r, z_r, dis_r, zs_r):
        d = d_r[...]
        disv = lax.rsqrt(1.0 + d[0] + d[1])
        dis_r[...] = disv
        zs_r[...] = z_r[...] * disv

    return pl.pallas_call(
        body,
        grid=(grid,),
        in_specs=[
            pl.BlockSpec((2, BLK, 1), lambda i: (0, i, 0)),
            pl.BlockSpec((BLK, F), lambda i: (i, 0)),
        ],
        out_specs=[
            pl.BlockSpec((BLK, 1), lambda i: (i, 0)),
            pl.BlockSpec((BLK, F), lambda i: (i, 0)),
        ],
        out_shape=[
            jax.ShapeDtypeStruct((N, 1), jnp.float32),
            jax.ShapeDtypeStruct((N, F), jnp.float32),
        ],
    )(deg_p[:, :, None], z1)


def _mid(p, zs1, dis2, b1, W3):
    N = zs1.shape[0]
    BLK = 1000
    grid = N // BLK
    hi = lax.Precision.HIGHEST

    def body(p_r, z_r, d_r, b_r, w_r, o_r):
        pv = p_r[...]
        disv = d_r[...]
        h = (pv[0] + pv[1] + z_r[...]) * disv + b_r[...]
        h = jnp.maximum(h, 0.0)
        z2 = jnp.dot(h, w_r[...],
                     preferred_element_type=jnp.float32, precision=hi)
        o_r[...] = z2 * disv

    return pl.pallas_call(
        body,
        grid=(grid,),
        in_specs=[
            pl.BlockSpec((NC, BLK, F), lambda i: (0, i, 0)),
            pl.BlockSpec((BLK, F), lambda i: (i, 0)),
            pl.BlockSpec((BLK, 1), lambda i: (i, 0)),
            pl.BlockSpec((F,), lambda i: (0,)),
            pl.BlockSpec((F, F), lambda i: (0, 0)),
        ],
        out_specs=pl.BlockSpec((BLK, F), lambda i: (i, 0)),
        out_shape=jax.ShapeDtypeStruct((N, F), jnp.float32),
    )(p, zs1, dis2, b1, W3)


def _fin(p, zs2, dis2, b3):
    N = zs2.shape[0]
    BLK = 1000
    grid = N // BLK

    def body(p_r, z_r, d_r, b_r, o_r):
        pv = p_r[...]
        o_r[...] = (pv[0] + pv[1] + z_r[...]) * d_r[...] + b_r[...]

    return pl.pallas_call(
        body,
        grid=(grid,),
        in_specs=[
            pl.BlockSpec((NC, BLK, F), lambda i: (0, i, 0)),
            pl.BlockSpec((BLK, F), lambda i: (i, 0)),
            pl.BlockSpec((BLK, 1), lambda i: (i, 0)),
            pl.BlockSpec((F,), lambda i: (0,)),
        ],
        out_specs=pl.BlockSpec((BLK, F), lambda i: (i, 0)),
        out_shape=jax.ShapeDtypeStruct((N, F), jnp.float32),
    )(p, zs2, dis2, b3)


def kernel(x, x_out, edge_index, edge_weight, c, W_aa, W_c, W_lm, b_lm,
           W1, b1, W3, b3):
    N, LM = x_out.shape
    E = edge_index.shape[1]
    src = edge_index[0]
    dst = edge_index[1]
    src2 = src.reshape(E // CH, CH)
    dst2 = dst.reshape(E // CH, CH)
    ew2 = edge_weight.reshape(E // CH, CH)

    degk = _make_deg(N, E)
    aggk = _make_agg(N, E)

    deg_p = degk(dst, edge_weight)
    z1 = _proj(x[:, None], c[:, None], x_out, W_aa, W_c, W_lm, b_lm, W1)
    dis2, zs1 = _dis(deg_p, z1)

    p1 = aggk(zs1, src2, dst2, ew2)
    zs2 = _mid(p1, zs1, dis2, b1, W3)
    p2 = aggk(zs2, src2, dst2, ew2)
    return _fin(p2, zs2, dis2, b3)
